# baseline (speedup 1.0000x reference)
import functools

import jax
import jax.numpy as jnp
from jax import lax
from jax.experimental import pallas as pl
from jax.experimental.pallas import tpu as pltpu
from jax.experimental.pallas import tpu_sc as plsc

B, NPATCH, DIM = 64, 576, 768
L = 577
LPW = 18
HALF = B // 2
NB = 4


def _sc_gather(pcol_flat, table):
    info = plsc.get_sparse_core_info()
    nw = info.num_cores * info.num_subcores
    mesh = plsc.VectorSubcoreMesh(core_axis_name="c", subcore_axis_name="s")

    @functools.partial(
        pl.kernel,
        mesh=mesh,
        out_type=jax.ShapeDtypeStruct((L, B, DIM), jnp.float32),
        scratch_types=[
            pltpu.VMEM((LPW * B,), jnp.int32),
            pltpu.VMEM((B,), jnp.int32),
            pltpu.VMEM((HALF, DIM), jnp.float32),
            pltpu.VMEM((HALF, DIM), jnp.float32),
            pltpu.VMEM((HALF, DIM), jnp.float32),
            pltpu.VMEM((HALF, DIM), jnp.float32),
            pltpu.SemaphoreType.DMA,
            pltpu.SemaphoreType.DMA,
            pltpu.SemaphoreType.DMA,
            pltpu.SemaphoreType.DMA,
            pltpu.SemaphoreType.DMA,
            pltpu.SemaphoreType.DMA,
            pltpu.SemaphoreType.DMA,
            pltpu.SemaphoreType.DMA,
        ],
    )
    def run(pcol_hbm, table_hbm, out_hbm, idxv, tidxv, r0, r1, r2, r3,
            g0, g1, g2, g3, w0, w1, w2, w3):
        wid = lax.axis_index("s") * info.num_cores + lax.axis_index("c")
        l0 = wid * LPW
        rings = (r0, r1, r2, r3)
        gsems = (g0, g1, g2, g3)
        wsems = (w0, w1, w2, w3)

        base = pl.multiple_of(wid * (LPW * B), LPW * B)
        pltpu.sync_copy(pcol_hbm.at[pl.ds(base, LPW * B)], idxv)

        nt = 2 * LPW
        g = [None] * nt
        w = [None] * nt

        def dst(t):
            return out_hbm.at[l0 + t // 2, pl.ds((t % 2) * HALF, HALF)]

        for t in range(nt):
            if t >= NB:
                w[t - NB].wait()
            g[t] = pltpu.async_copy(
                table_hbm.at[idxv.at[pl.ds(t * HALF, HALF)]],
                rings[t % NB],
                gsems[t % NB],
            )
            if t >= 1:
                g[t - 1].wait()
                w[t - 1] = pltpu.async_copy(
                    rings[(t - 1) % NB], dst(t - 1), wsems[(t - 1) % NB]
                )
        g[nt - 1].wait()
        w[nt - 1] = pltpu.async_copy(
            rings[(nt - 1) % NB], dst(nt - 1), wsems[(nt - 1) % NB]
        )
        for t in range(nt - NB, nt - 1):
            w[t].wait()
        @pl.when(wid == nw - 1)
        def _tail():
            pltpu.sync_copy(pcol_hbm.at[pl.ds(nw * LPW * B, B)], tidxv)
            ga = pltpu.async_copy(table_hbm.at[tidxv.at[pl.ds(0, HALF)]], r0, g0)
            gb = pltpu.async_copy(table_hbm.at[tidxv.at[pl.ds(HALF, HALF)]], r1, g1)
            ga.wait()
            wa = pltpu.async_copy(r0, out_hbm.at[nw * LPW, pl.ds(0, HALF)], w0)
            gb.wait()
            wb = pltpu.async_copy(r1, out_hbm.at[nw * LPW, pl.ds(HALF, HALF)], w1)
            wa.wait()
            wb.wait()
        w[nt - 1].wait()

    return run(pcol_flat, table)


def kernel(pid, pos_embeds):
    p = jnp.pad(pid.astype(jnp.int32).T + 1, ((1, 0), (0, 0)))
    out_t = _sc_gather(p.reshape(-1), pos_embeds)
    return jnp.transpose(out_t, (1, 0, 2))


# speedup vs baseline: 1.0196x; 1.0196x over previous
import functools

import jax
import jax.numpy as jnp
from jax import lax
from jax.experimental import pallas as pl
from jax.experimental.pallas import tpu as pltpu
from jax.experimental.pallas import tpu_sc as plsc

B, NPATCH, DIM = 64, 576, 768
L = 577
LPW = 18
RPW = LPW * B
CHUNK = 72
NCH = RPW // CHUNK


def _sc_gather(pcol_flat, table):
    info = plsc.get_sparse_core_info()
    nw = info.num_cores * info.num_subcores
    mesh = plsc.VectorSubcoreMesh(core_axis_name="c", subcore_axis_name="s")

    @functools.partial(
        pl.kernel,
        mesh=mesh,
        out_type=jax.ShapeDtypeStruct((L * B, DIM), jnp.float32),
        scratch_types=[
            pltpu.VMEM((RPW,), jnp.int32),
            pltpu.VMEM((B,), jnp.int32),
            pltpu.VMEM((CHUNK, DIM), jnp.float32),
            pltpu.VMEM((CHUNK, DIM), jnp.float32),
            pltpu.SemaphoreType.DMA,
            pltpu.SemaphoreType.DMA,
            pltpu.SemaphoreType.DMA,
            pltpu.SemaphoreType.DMA,
        ],
    )
    def run(pcol_hbm, table_hbm, out_hbm, idxv, tidxv, rows0, rows1,
            gs0, gs1, ws0, ws1):
        wid = lax.axis_index("s") * info.num_cores + lax.axis_index("c")
        rows, gsems, wsems = (rows0, rows1), (gs0, gs1), (ws0, ws1)

        base = pl.multiple_of(wid * RPW, RPW)
        pltpu.sync_copy(pcol_hbm.at[pl.ds(base, RPW)], idxv)

        g = [None] * NCH
        w = [None] * NCH
        for t in range(NCH):
            if t >= 2:
                w[t - 2].wait()
            g[t] = pltpu.async_copy(
                table_hbm.at[idxv.at[pl.ds(t * CHUNK, CHUNK)]],
                rows[t % 2],
                gsems[t % 2],
            )
            if t >= 1:
                g[t - 1].wait()
                w[t - 1] = pltpu.async_copy(
                    rows[(t - 1) % 2],
                    out_hbm.at[pl.ds(base + (t - 1) * CHUNK, CHUNK)],
                    wsems[(t - 1) % 2],
                )
        g[NCH - 1].wait()
        w[NCH - 1] = pltpu.async_copy(
            rows[(NCH - 1) % 2],
            out_hbm.at[pl.ds(base + (NCH - 1) * CHUNK, CHUNK)],
            wsems[(NCH - 1) % 2],
        )
        w[NCH - 2].wait()
        @pl.when(wid == nw - 1)
        def _tail():
            pltpu.sync_copy(pcol_hbm.at[pl.ds(nw * RPW, B)], tidxv)
            pltpu.async_copy(table_hbm.at[tidxv], rows0.at[pl.ds(0, B)], gs0).wait()
            pltpu.sync_copy(rows0.at[pl.ds(0, B)], out_hbm.at[pl.ds(nw * RPW, B)])
        w[NCH - 1].wait()

    return run(pcol_flat, table)


def kernel(pid, pos_embeds):
    p = jnp.pad(pid.astype(jnp.int32).T + 1, ((1, 0), (0, 0)))
    out_flat = _sc_gather(p.reshape(-1), pos_embeds)
    return jnp.transpose(out_flat.reshape(L, B, DIM), (1, 0, 2))


# speedup vs baseline: 1.0230x; 1.0033x over previous
import functools

import jax
import jax.numpy as jnp
from jax import lax
from jax.experimental import pallas as pl
from jax.experimental.pallas import tpu as pltpu
from jax.experimental.pallas import tpu_sc as plsc

B, NPATCH, DIM = 64, 576, 768
L = 577
LPW = 18


def _sc_gather(pcol_flat, table):
    info = plsc.get_sparse_core_info()
    nw = info.num_cores * info.num_subcores
    mesh = plsc.VectorSubcoreMesh(core_axis_name="c", subcore_axis_name="s")

    @functools.partial(
        pl.kernel,
        mesh=mesh,
        out_type=jax.ShapeDtypeStruct((L, B, DIM), jnp.float32),
        scratch_types=[
            pltpu.VMEM((LPW * B,), jnp.int32),
            pltpu.VMEM((B,), jnp.int32),
            pltpu.VMEM((B, DIM), jnp.float32),
            pltpu.VMEM((B, DIM), jnp.float32),
            pltpu.SemaphoreType.DMA,
            pltpu.SemaphoreType.DMA,
            pltpu.SemaphoreType.DMA,
            pltpu.SemaphoreType.DMA,
        ],
    )
    def run(pcol_hbm, table_hbm, out_hbm, idxv, tidxv, rows0, rows1,
            gs0, gs1, ws0, ws1):
        wid = lax.axis_index("s") * info.num_cores + lax.axis_index("c")
        l0 = wid * LPW
        rows, gsems, wsems = (rows0, rows1), (gs0, gs1), (ws0, ws1)

        base = pl.multiple_of(wid * (LPW * B), LPW * B)
        pltpu.sync_copy(pcol_hbm.at[pl.ds(base, LPW * B)], idxv)

        g = [None] * LPW
        w = [None] * LPW
        for t in range(LPW):
            if t >= 2:
                w[t - 2].wait()
            g[t] = pltpu.async_copy(
                table_hbm.at[idxv.at[pl.ds(t * B, B)]], rows[t % 2], gsems[t % 2]
            )
            if t >= 1:
                g[t - 1].wait()
                w[t - 1] = pltpu.async_copy(
                    rows[(t - 1) % 2], out_hbm.at[l0 + t - 1], wsems[(t - 1) % 2]
                )
        g[LPW - 1].wait()
        w[LPW - 1] = pltpu.async_copy(
            rows[(LPW - 1) % 2], out_hbm.at[l0 + LPW - 1], wsems[(LPW - 1) % 2]
        )
        w[LPW - 2].wait()
        @pl.when(wid == nw - 1)
        def _tail():
            pltpu.sync_copy(pcol_hbm.at[pl.ds(nw * LPW * B, B)], tidxv)
            pltpu.async_copy(table_hbm.at[tidxv], rows0, gs0).wait()
            pltpu.sync_copy(rows0, out_hbm.at[nw * LPW])
        w[LPW - 1].wait()

    return run(pcol_flat, table)


def kernel(pid, pos_embeds):
    p = jnp.pad(pid.astype(jnp.int32).T + 1, ((1, 0), (0, 0)))
    out_t = _sc_gather(p.reshape(-1), pos_embeds)
    return jnp.transpose(out_t, (1, 0, 2))
